# baseline (speedup 1.0000x reference)
import functools

import jax
import jax.numpy as jnp
from jax import lax
from jax.experimental import pallas as pl
from jax.experimental.pallas import tpu as pltpu
from jax.experimental.pallas import tpu_sc as plsc

NC = 2
NS = 16
NW = NC * NS

B = 4096
L = 200
D = 64
TOTAL = B * L
PER_W = TOTAL // NW
G = 128
CHUNKS = PER_W // G


def _gather_kernel(table_hbm, idx_hbm, out_hbm, idx_v, rows_v, sem):
    c = lax.axis_index("c")
    s = lax.axis_index("s")
    wid = s * NC + c
    pltpu.sync_copy(idx_hbm.at[wid], idx_v)
    base = wid * PER_W

    def body(j, carry):
        pltpu.async_copy(table_hbm.at[idx_v.at[j]], rows_v, sem).wait()
        pltpu.sync_copy(rows_v, out_hbm.at[pl.ds(base + j * G, G)])
        return carry

    lax.fori_loop(0, CHUNKS, body, 0)


@jax.jit
def _run(table, idx3):
    k = functools.partial(
        pl.kernel,
        out_type=jax.ShapeDtypeStruct((TOTAL, D), jnp.float32),
        mesh=plsc.VectorSubcoreMesh(core_axis_name="c", subcore_axis_name="s"),
        scratch_types=[
            pltpu.VMEM((CHUNKS, G), jnp.int32),
            pltpu.VMEM((G, D), jnp.float32),
            pltpu.SemaphoreType.DMA,
        ],
        compiler_params=pltpu.CompilerParams(use_tc_tiling_on_sc=False),
    )(_gather_kernel)
    return k(table, idx3)


def kernel(batch, table):
    idx3 = batch.reshape(NW, CHUNKS, G)
    out = _run(table, idx3)
    return out.reshape(B, L, D)


# speedup vs baseline: 1.1163x; 1.1163x over previous
import functools

import jax
import jax.numpy as jnp
from jax import lax
from jax.experimental import pallas as pl
from jax.experimental.pallas import tpu as pltpu
from jax.experimental.pallas import tpu_sc as plsc

NC = 2
NS = 16
NW = NC * NS

B = 4096
L = 200
D = 64
TOTAL = B * L
PER_W = TOTAL // NW
G = 128
CHUNKS = PER_W // G


R = 8
FD = 6


def _gather_kernel(table_hbm, idx_hbm, out_hbm, idx_v, rows_v, gsem, osem):
    c = lax.axis_index("c")
    s = lax.axis_index("s")
    wid = s * NC + c
    pltpu.sync_copy(idx_hbm.at[wid], idx_v)
    base = wid * PER_W

    def fire_gather(j, slot):
        pltpu.async_copy(table_hbm.at[idx_v.at[j]], rows_v.at[slot],
                         gsem.at[slot])

    def wait_gather(slot):
        pltpu.make_async_copy(table_hbm.at[idx_v.at[0]], rows_v.at[slot],
                              gsem.at[slot]).wait()

    def fire_out(j, slot):
        pltpu.async_copy(rows_v.at[slot], out_hbm.at[pl.ds(base + j * G, G)],
                         osem.at[slot])

    def wait_out(slot):
        pltpu.make_async_copy(rows_v.at[slot], out_hbm.at[pl.ds(base, G)],
                              osem.at[slot]).wait()

    for p in range(FD):
        fire_gather(p, p)

    def body(j, carry):
        slot = lax.rem(j, R)
        wait_gather(slot)
        fire_out(j, slot)

        @pl.when(j < CHUNKS - FD)
        def _fire_next():
            f = j + FD
            slot2 = lax.rem(f, R)

            @pl.when(f >= R)
            def _recycle():
                wait_out(slot2)

            fire_gather(f, slot2)

        return carry

    lax.fori_loop(0, CHUNKS, body, 0)

    for p in range(R):
        wait_out((CHUNKS - R + p) % R)


@jax.jit
def _run(table, idx3):
    k = functools.partial(
        pl.kernel,
        out_type=jax.ShapeDtypeStruct((TOTAL, D), jnp.float32),
        mesh=plsc.VectorSubcoreMesh(core_axis_name="c", subcore_axis_name="s"),
        scratch_types=[
            pltpu.VMEM((CHUNKS, G), jnp.int32),
            pltpu.VMEM((R, G, D), jnp.float32),
            pltpu.SemaphoreType.DMA((R,)),
            pltpu.SemaphoreType.DMA((R,)),
        ],
        compiler_params=pltpu.CompilerParams(use_tc_tiling_on_sc=False),
    )(_gather_kernel)
    return k(table, idx3)


def kernel(batch, table):
    idx3 = batch.reshape(NW, CHUNKS, G)
    out = _run(table, idx3)
    return out.reshape(B, L, D)
